# Initial kernel scaffold; baseline (speedup 1.0000x reference)
import jax
import jax.numpy as jnp
from jax.experimental import pallas as pl


def kernel(X, L_u, L_d, W1u, W1d, W1i, W2u, W2d, W2i, W3u, W3d, W3i, Wfc):
    raise NotImplementedError("write your pallas kernel here")


# speedup vs baseline: 1.0304x; 1.0304x over previous
import jax
import jax.numpy as jnp
from jax.experimental import pallas as pl
from jax.experimental.pallas import tpu as pltpu


def _dot(a, b):
    return jnp.dot(a, b, preferred_element_type=jnp.float32)


def _proj_kernel(x_ref, wu_ref, wd_ref, wi_ref, yu_ref, yd_ref, yi_ref):
    x = x_ref[...]
    yu_ref[...] = _dot(x, wu_ref[...])
    yd_ref[...] = _dot(x, wd_ref[...])
    yi_ref[...] = _dot(x, wi_ref[...])


def _layer_kernel(lu_ref, ld_ref, yu_ref, yd_ref, yi_ref,
                  wu_ref, wd_ref, wi_ref,
                  nyu_ref, nyd_ref, nyi_ref):
    z = jnp.tanh(_dot(lu_ref[...], yu_ref[...])
                 + _dot(ld_ref[...], yd_ref[...])
                 + yi_ref[...])
    nyu_ref[...] = _dot(z, wu_ref[...])
    nyd_ref[...] = _dot(z, wd_ref[...])
    nyi_ref[...] = _dot(z, wi_ref[...])


def _final_kernel(lu_ref, ld_ref, yu_ref, yd_ref, yi_ref, wfc_ref, out_ref):
    z = jnp.tanh(_dot(lu_ref[...], yu_ref[...])
                 + _dot(ld_ref[...], yd_ref[...])
                 + yi_ref[...])
    g = _dot(z, wfc_ref[...])
    nrm = jnp.sqrt(jnp.sum(g * g, axis=1, keepdims=True))
    nrm = jnp.maximum(nrm, 1e-12)
    out_ref[...] = jnp.tanh(g / nrm)


def kernel(X, L_u, L_d, W1u, W1d, W1i, W2u, W2d, W2i, W3u, W3d, W3i, Wfc):
    n, f = X.shape
    h = W1u.shape[1]
    o = Wfc.shape[1]
    bm = 200 if n % 200 == 0 else n
    it = n // bm

    whole = lambda shape: pl.BlockSpec(shape, lambda i: (0, 0))
    row_f = pl.BlockSpec((bm, h), lambda i: (i, 0))
    l_stripe = pl.BlockSpec((bm, n), lambda i: (i, 0))
    params = pltpu.CompilerParams(dimension_semantics=("parallel",))

    proj = pl.pallas_call(
        _proj_kernel,
        grid=(it,),
        in_specs=[
            pl.BlockSpec((bm, f), lambda i: (i, 0)),
            whole((f, h)), whole((f, h)), whole((f, h)),
        ],
        out_specs=[row_f] * 3,
        out_shape=[jax.ShapeDtypeStruct((n, h), jnp.float32)] * 3,
        compiler_params=params,
    )
    yu, yd, yi = proj(X, W1u, W1d, W1i)

    def layer(yu, yd, yi, wu, wd, wi):
        return pl.pallas_call(
            _layer_kernel,
            grid=(it,),
            in_specs=[
                l_stripe, l_stripe,
                whole((n, h)), whole((n, h)), row_f,
                whole((h, h)), whole((h, h)), whole((h, h)),
            ],
            out_specs=[row_f] * 3,
            out_shape=[jax.ShapeDtypeStruct((n, h), jnp.float32)] * 3,
            compiler_params=params,
        )(L_u, L_d, yu, yd, yi, wu, wd, wi)

    yu, yd, yi = layer(yu, yd, yi, W2u, W2d, W2i)
    yu, yd, yi = layer(yu, yd, yi, W3u, W3d, W3i)

    out = pl.pallas_call(
        _final_kernel,
        grid=(it,),
        in_specs=[
            l_stripe, l_stripe,
            whole((n, h)), whole((n, h)), row_f,
            whole((h, o)),
        ],
        out_specs=pl.BlockSpec((bm, o), lambda i: (i, 0)),
        out_shape=jax.ShapeDtypeStruct((n, o), jnp.float32),
        compiler_params=params,
    )(L_u, L_d, yu, yd, yi, Wfc)
    return out
